# baseline (speedup 1.0000x reference)
import functools

import jax
import jax.numpy as jnp
from jax import lax
from jax.experimental import pallas as pl
from jax.experimental.pallas import tpu as pltpu
from jax.experimental.pallas import tpu_sc as plsc

N = 10000
E = 320000
D = 128

NC, NS, L = 2, 16, 16
NW = NC * NS
CH = 128

S = 5
SE = E // S
EPW = SE // NW
FCH = EPW // CH
TAIL = EPW - FCH * CH
NCHS = SE // CH
FCHR = NCHS // NW
TAILC = NCHS - FCHR * NW

BN = 2000
BE = 2000

NPAD = 10240
NPT = NPAD // NS
ZR = 128

_mesh = plsc.VectorSubcoreMesh(core_axis_name="c", subcore_axis_name="s")


def _pre_body(x_ref, wa_ref, wb_ref, ua_ref, b1_ref, ub1_ref,
              xa_ref, xb_ref, xu_ref):
    x = x_ref[...]
    xa_ref[...] = jnp.dot(x, wa_ref[...], preferred_element_type=jnp.float32) + b1_ref[...]
    xb_ref[...] = jnp.dot(x, wb_ref[...], preferred_element_type=jnp.float32)
    xu_ref[...] = jnp.dot(x, ua_ref[...], preferred_element_type=jnp.float32) + ub1_ref[...]


def _precompute(x, wa, wb, ua, b1, ub1):
    grid = (N // BN,)
    return pl.pallas_call(
        _pre_body,
        grid=grid,
        in_specs=[
            pl.BlockSpec((BN, D), lambda i: (i, 0)),
            pl.BlockSpec((D, D), lambda i: (0, 0)),
            pl.BlockSpec((D, D), lambda i: (0, 0)),
            pl.BlockSpec((D, D), lambda i: (0, 0)),
            pl.BlockSpec((1, D), lambda i: (0, 0)),
            pl.BlockSpec((1, D), lambda i: (0, 0)),
        ],
        out_specs=[
            pl.BlockSpec((BN, D), lambda i: (i, 0)),
            pl.BlockSpec((BN, D), lambda i: (i, 0)),
            pl.BlockSpec((BN, D), lambda i: (i, 0)),
        ],
        out_shape=[jax.ShapeDtypeStruct((N, D), jnp.float32)] * 3,
    )(x, wa, wb, ua, b1, ub1)


def _vadd_rows(dst_buf, src_buf, nrows):
    def row(r, carry):
        for l in range(D // L):
            sl = pl.ds(l * L, L)
            dst_buf[r, sl] = dst_buf[r, sl] + src_buf[r, sl]
        return carry

    lax.fori_loop(0, nrows, row, 0)


@functools.partial(
    pl.kernel,
    out_type=jax.ShapeDtypeStruct((SE, D), jnp.float32),
    mesh=_mesh,
    scratch_types=[
        pltpu.VMEM((EPW,), jnp.int32),
        pltpu.VMEM((EPW,), jnp.int32),
        pltpu.VMEM((CH, D), jnp.float32),
        pltpu.VMEM((CH, D), jnp.float32),
        pltpu.VMEM((CH, D), jnp.float32),
        pltpu.VMEM((CH, D), jnp.float32),
        pltpu.SemaphoreType.DMA,
        pltpu.SemaphoreType.DMA,
        pltpu.SemaphoreType.DMA,
        pltpu.SemaphoreType.DMA,
        pltpu.SemaphoreType.DMA,
        pltpu.SemaphoreType.DMA,
    ],
)
def _gather(xa_hbm, xb_hbm, dst_hbm, src_hbm, g_hbm,
            idxd, idxs, bufa0, bufa1, bufb0, bufb1,
            sa0, sa1, sb0, sb1, sw0, sw1):
    w = lax.axis_index("s") * NC + lax.axis_index("c")
    base = w * EPW
    bufa = (bufa0, bufa1)
    bufb = (bufb0, bufb1)
    sa = (sa0, sa1)
    sb = (sb0, sb1)
    sw = (sw0, sw1)

    pltpu.sync_copy(dst_hbm.at[pl.ds(base, EPW)], idxd)
    pltpu.sync_copy(src_hbm.at[pl.ds(base, EPW)], idxs)

    def issue_gathers(t, b):
        off = t * CH
        pltpu.async_copy(xa_hbm.at[idxd.at[pl.ds(off, CH)]], bufa[b], sa[b])
        pltpu.async_copy(xb_hbm.at[idxs.at[pl.ds(off, CH)]], bufb[b], sb[b])

    def wait_gathers(b):
        pltpu.make_async_copy(xa_hbm.at[idxd.at[pl.ds(0, CH)]], bufa[b], sa[b]).wait()
        pltpu.make_async_copy(xb_hbm.at[idxs.at[pl.ds(0, CH)]], bufb[b], sb[b]).wait()

    def wait_write(b):
        pltpu.make_async_copy(bufa[b], g_hbm.at[pl.ds(base, CH)], sw[b]).wait()

    def process(t, b):
        wait_gathers(b)
        _vadd_rows(bufa[b], bufb[b], CH)
        pltpu.async_copy(bufa[b], g_hbm.at[pl.ds(base + t * CH, CH)], sw[b])

    issue_gathers(0, 0)

    def body(i, carry):
        for b in (0, 1):
            t = 2 * i + b
            nb = 1 - b

            @pl.when(t + 1 < FCH)
            def _():
                @pl.when(t >= 1)
                def _():
                    wait_write(nb)
                issue_gathers(t + 1, nb)

            process(t, b)
        return carry

    lax.fori_loop(0, FCH // 2, body, 0)
    if FCH % 2:
        process(FCH - 1, (FCH - 1) % 2)

    wait_write(0)
    wait_write(1)

    toff = FCH * CH
    ta = pltpu.async_copy(
        xa_hbm.at[idxd.at[pl.ds(toff, TAIL)]], bufa0.at[pl.ds(0, TAIL)], sa0)
    tb = pltpu.async_copy(
        xb_hbm.at[idxs.at[pl.ds(toff, TAIL)]], bufb0.at[pl.ds(0, TAIL)], sb0)
    ta.wait()
    tb.wait()
    _vadd_rows(bufa0, bufb0, TAIL)
    pltpu.sync_copy(bufa0.at[pl.ds(0, TAIL)], g_hbm.at[pl.ds(base + toff, TAIL)])


def _mlp_body(g_ref, ea_ref, w1c_ref, w2_ref, b2_ref, m_ref):
    m1 = g_ref[...] + jnp.dot(
        ea_ref[...], w1c_ref[...], preferred_element_type=jnp.float32)
    m_ref[...] = jnp.dot(jnp.maximum(m1, 0.0), w2_ref[...],
                         preferred_element_type=jnp.float32) + b2_ref[...]


def _edge_mlp(g, ea, w1c, w2, b2):
    grid = (SE // BE,)
    return pl.pallas_call(
        _mlp_body,
        grid=grid,
        in_specs=[
            pl.BlockSpec((BE, D), lambda i: (i, 0)),
            pl.BlockSpec((BE, D), lambda i: (i, 0)),
            pl.BlockSpec((D, D), lambda i: (0, 0)),
            pl.BlockSpec((D, D), lambda i: (0, 0)),
            pl.BlockSpec((1, D), lambda i: (0, 0)),
        ],
        out_specs=pl.BlockSpec((BE, D), lambda i: (i, 0)),
        out_shape=jax.ShapeDtypeStruct((SE, D), jnp.float32),
    )(g, ea, w1c, w2, b2)


@functools.partial(
    pl.kernel,
    out_type=jax.ShapeDtypeStruct((NC, NPAD, D), jnp.float32),
    mesh=_mesh,
    scratch_types=[
        pltpu.VMEM((CH,), jnp.int32),
        pltpu.VMEM((CH,), jnp.int32),
        pltpu.VMEM((CH, D), jnp.float32),
        pltpu.VMEM((CH, D), jnp.float32),
        pltpu.VMEM_SHARED((NPAD, D), jnp.float32),
        pltpu.SemaphoreType.DMA,
        pltpu.SemaphoreType.DMA,
        pltpu.SemaphoreType.DMA,
        pltpu.SemaphoreType.DMA,
        pltpu.SemaphoreType.DMA,
        pltpu.SemaphoreType.DMA,
    ],
)
def _scatter(m_hbm, dst2d_hbm, out_hbm,
             idx0, idx1, rows0, rows1, accum,
             si0, si1, sr0, sr1, ss0, ss1):
    cid = lax.axis_index("c")
    sid = lax.axis_index("s")
    w = sid * NC + cid
    idxv = (idx0, idx1)
    rows = (rows0, rows1)
    si = (si0, si1)
    sr = (sr0, sr1)
    ss = (ss0, ss1)

    def zrow(i, carry):
        for l in range(D // L):
            rows0[i, pl.ds(l * L, L)] = jnp.zeros((L,), jnp.float32)
        return carry

    lax.fori_loop(0, ZR, zrow, 0)
    for k in range(NPT // ZR):
        pltpu.sync_copy(rows0, accum.at[pl.ds(sid * NPT + k * ZR, ZR)])
    plsc.subcore_barrier()

    def issue_loads(c, b):
        pltpu.async_copy(dst2d_hbm.at[c], idxv[b], si[b])
        pltpu.async_copy(m_hbm.at[pl.ds(c * CH, CH)], rows[b], sr[b])

    def wait_loads(b):
        pltpu.make_async_copy(dst2d_hbm.at[0], idxv[b], si[b]).wait()
        pltpu.make_async_copy(m_hbm.at[pl.ds(0, CH)], rows[b], sr[b]).wait()

    def wait_scatter(b):
        pltpu.make_async_copy(rows[b], accum.at[idxv[b]], ss[b]).wait()

    issue_loads(w, 0)

    def body(i, carry):
        for b in (0, 1):
            t = 2 * i + b
            nb = 1 - b
            c = w + t * NW

            @pl.when(t + 1 < FCHR)
            def _():
                @pl.when(t >= 1)
                def _():
                    wait_scatter(nb)
                issue_loads(c + NW, nb)

            wait_loads(b)
            pltpu.async_copy(rows[b], accum.at[idxv[b]], ss[b], add=True)
        return carry

    lax.fori_loop(0, FCHR // 2, body, 0)
    if FCHR % 2:
        b = (FCHR - 1) % 2
        wait_loads(b)
        pltpu.async_copy(rows[b], accum.at[idxv[b]], ss[b], add=True)
    wait_scatter(0)
    wait_scatter(1)

    @pl.when(w < TAILC)
    def _():
        c = w + FCHR * NW
        issue_loads(c, 0)
        wait_loads(0)
        pltpu.sync_copy(rows0, accum.at[idx0], add=True)

    plsc.subcore_barrier()

    for k in range(NPT // ZR):
        r0 = sid * NPT + k * ZR
        pltpu.sync_copy(accum.at[pl.ds(r0, ZR)], rows0)
        pltpu.sync_copy(rows0, out_hbm.at[cid, pl.ds(r0, ZR)])


def _upd_body(*refs):
    xu_ref = refs[0]
    p_refs = refs[1:1 + 2 * S]
    u1b_ref, u2_ref, ub2_ref, out_ref = refs[1 + 2 * S:]
    aggr = p_refs[0][...]
    for r in p_refs[1:]:
        aggr = aggr + r[...]
    h = jnp.maximum(
        xu_ref[...] + jnp.dot(aggr, u1b_ref[...],
                              preferred_element_type=jnp.float32), 0.0)
    out_ref[...] = jnp.dot(h, u2_ref[...],
                           preferred_element_type=jnp.float32) + ub2_ref[...]


def _update(xu, parts, u1b, u2, ub2):
    grid = (N // BN,)
    nd = pl.BlockSpec((BN, D), lambda i: (i, 0))
    wspec = pl.BlockSpec((D, D), lambda i: (0, 0))
    bspec = pl.BlockSpec((1, D), lambda i: (0, 0))
    return pl.pallas_call(
        _upd_body,
        grid=grid,
        in_specs=[nd] + [nd] * (2 * S) + [wspec, wspec, bspec],
        out_specs=nd,
        out_shape=jax.ShapeDtypeStruct((N, D), jnp.float32),
    )(xu, *parts, u1b, u2, ub2)


def kernel(x, edge_index, edge_attr, W1, b1, W2, b2, U1, ub1, U2, ub2):
    src = edge_index[0].astype(jnp.int32)
    dst = edge_index[1].astype(jnp.int32)
    wa, wb, w1c = W1[0:D], W1[D:2 * D], W1[2 * D:]
    ua, u1b = U1[0:D], U1[D:]
    b1r = b1.reshape(1, D)
    b2r = b2.reshape(1, D)
    ub1r = ub1.reshape(1, D)
    ub2r = ub2.reshape(1, D)

    xa, xb, xu = _precompute(x, wa, wb, ua, b1r, ub1r)

    parts = []
    for s in range(S):
        lo, hi = s * SE, (s + 1) * SE
        dst_s = dst[lo:hi]
        src_s = src[lo:hi]
        g = _gather(xa, xb, dst_s, src_s)
        m = _edge_mlp(g, edge_attr[lo:hi], w1c, W2, b2r)
        p = _scatter(m, dst_s.reshape(NCHS, CH))
        parts.append(p[0])
        parts.append(p[1])

    return _update(xu, parts, u1b, U2, ub2r)


# speedup vs baseline: 1.2376x; 1.2376x over previous
import functools

import jax
import jax.numpy as jnp
from jax import lax
from jax.experimental import pallas as pl
from jax.experimental.pallas import tpu as pltpu
from jax.experimental.pallas import tpu_sc as plsc

N = 10000
E = 320000
D = 128

NC, NS, L = 2, 16, 16
NW = NC * NS
CH = 128

S = 2
SE = E // S
EPW = SE // NW
FCH = EPW // CH
TAIL = EPW - FCH * CH
NCHS = SE // CH
FCHR = NCHS // NW
TAILC = NCHS - FCHR * NW

BN = 2000
BE = 2000

NPAD = 10240
NPT = NPAD // NS
ZR = 128

_mesh = plsc.VectorSubcoreMesh(core_axis_name="c", subcore_axis_name="s")


def _pre_body(x_ref, wa_ref, wb_ref, ua_ref, b1_ref, ub1_ref,
              xa_ref, xb_ref, xu_ref):
    x = x_ref[...]
    xa_ref[...] = jnp.dot(x, wa_ref[...], preferred_element_type=jnp.float32) + b1_ref[...]
    xb_ref[...] = jnp.dot(x, wb_ref[...], preferred_element_type=jnp.float32)
    xu_ref[...] = jnp.dot(x, ua_ref[...], preferred_element_type=jnp.float32) + ub1_ref[...]


def _precompute(x, wa, wb, ua, b1, ub1):
    grid = (N // BN,)
    return pl.pallas_call(
        _pre_body,
        grid=grid,
        in_specs=[
            pl.BlockSpec((BN, D), lambda i: (i, 0)),
            pl.BlockSpec((D, D), lambda i: (0, 0)),
            pl.BlockSpec((D, D), lambda i: (0, 0)),
            pl.BlockSpec((D, D), lambda i: (0, 0)),
            pl.BlockSpec((1, D), lambda i: (0, 0)),
            pl.BlockSpec((1, D), lambda i: (0, 0)),
        ],
        out_specs=[
            pl.BlockSpec((BN, D), lambda i: (i, 0)),
            pl.BlockSpec((BN, D), lambda i: (i, 0)),
            pl.BlockSpec((BN, D), lambda i: (i, 0)),
        ],
        out_shape=[jax.ShapeDtypeStruct((N, D), jnp.float32)] * 3,
    )(x, wa, wb, ua, b1, ub1)


def _vadd_rows(dst_buf, src_buf, nrows):
    def row(r, carry):
        for l in range(D // L):
            sl = pl.ds(l * L, L)
            dst_buf[r, sl] = dst_buf[r, sl] + src_buf[r, sl]
        return carry

    lax.fori_loop(0, nrows, row, 0)


@functools.partial(
    pl.kernel,
    out_type=jax.ShapeDtypeStruct((SE, D), jnp.float32),
    mesh=_mesh,
    scratch_types=[
        pltpu.VMEM((EPW,), jnp.int32),
        pltpu.VMEM((EPW,), jnp.int32),
        pltpu.VMEM((CH, D), jnp.float32),
        pltpu.VMEM((CH, D), jnp.float32),
        pltpu.VMEM((CH, D), jnp.float32),
        pltpu.VMEM((CH, D), jnp.float32),
        pltpu.SemaphoreType.DMA,
        pltpu.SemaphoreType.DMA,
        pltpu.SemaphoreType.DMA,
        pltpu.SemaphoreType.DMA,
        pltpu.SemaphoreType.DMA,
        pltpu.SemaphoreType.DMA,
    ],
)
def _gather(xa_hbm, xb_hbm, dst_hbm, src_hbm, g_hbm,
            idxd, idxs, bufa0, bufa1, bufb0, bufb1,
            sa0, sa1, sb0, sb1, sw0, sw1):
    w = lax.axis_index("s") * NC + lax.axis_index("c")
    base = w * EPW
    bufa = (bufa0, bufa1)
    bufb = (bufb0, bufb1)
    sa = (sa0, sa1)
    sb = (sb0, sb1)
    sw = (sw0, sw1)

    pltpu.sync_copy(dst_hbm.at[pl.ds(base, EPW)], idxd)
    pltpu.sync_copy(src_hbm.at[pl.ds(base, EPW)], idxs)

    def issue_gathers(t, b):
        off = t * CH
        pltpu.async_copy(xa_hbm.at[idxd.at[pl.ds(off, CH)]], bufa[b], sa[b])
        pltpu.async_copy(xb_hbm.at[idxs.at[pl.ds(off, CH)]], bufb[b], sb[b])

    def wait_gathers(b):
        pltpu.make_async_copy(xa_hbm.at[idxd.at[pl.ds(0, CH)]], bufa[b], sa[b]).wait()
        pltpu.make_async_copy(xb_hbm.at[idxs.at[pl.ds(0, CH)]], bufb[b], sb[b]).wait()

    def wait_write(b):
        pltpu.make_async_copy(bufa[b], g_hbm.at[pl.ds(base, CH)], sw[b]).wait()

    def process(t, b):
        wait_gathers(b)
        _vadd_rows(bufa[b], bufb[b], CH)
        pltpu.async_copy(bufa[b], g_hbm.at[pl.ds(base + t * CH, CH)], sw[b])

    issue_gathers(0, 0)

    def body(i, carry):
        for b in (0, 1):
            t = 2 * i + b
            nb = 1 - b

            @pl.when(t + 1 < FCH)
            def _():
                @pl.when(t >= 1)
                def _():
                    wait_write(nb)
                issue_gathers(t + 1, nb)

            process(t, b)
        return carry

    lax.fori_loop(0, FCH // 2, body, 0)
    if FCH % 2:
        process(FCH - 1, (FCH - 1) % 2)

    wait_write(0)
    wait_write(1)

    toff = FCH * CH
    ta = pltpu.async_copy(
        xa_hbm.at[idxd.at[pl.ds(toff, TAIL)]], bufa0.at[pl.ds(0, TAIL)], sa0)
    tb = pltpu.async_copy(
        xb_hbm.at[idxs.at[pl.ds(toff, TAIL)]], bufb0.at[pl.ds(0, TAIL)], sb0)
    ta.wait()
    tb.wait()
    _vadd_rows(bufa0, bufb0, TAIL)
    pltpu.sync_copy(bufa0.at[pl.ds(0, TAIL)], g_hbm.at[pl.ds(base + toff, TAIL)])


def _mlp_body(g_ref, ea_ref, w1c_ref, w2_ref, b2_ref, m_ref):
    m1 = g_ref[...] + jnp.dot(
        ea_ref[...], w1c_ref[...], preferred_element_type=jnp.float32)
    m_ref[...] = jnp.dot(jnp.maximum(m1, 0.0), w2_ref[...],
                         preferred_element_type=jnp.float32) + b2_ref[...]


def _edge_mlp(g, ea, w1c, w2, b2, slab):
    grid = (SE // BE,)
    off = slab * (SE // BE)
    return pl.pallas_call(
        _mlp_body,
        grid=grid,
        in_specs=[
            pl.BlockSpec((BE, D), lambda i: (i, 0)),
            pl.BlockSpec((BE, D), lambda i: (off + i, 0)),
            pl.BlockSpec((D, D), lambda i: (0, 0)),
            pl.BlockSpec((D, D), lambda i: (0, 0)),
            pl.BlockSpec((1, D), lambda i: (0, 0)),
        ],
        out_specs=pl.BlockSpec((BE, D), lambda i: (i, 0)),
        out_shape=jax.ShapeDtypeStruct((SE, D), jnp.float32),
    )(g, ea, w1c, w2, b2)


@functools.partial(
    pl.kernel,
    out_type=jax.ShapeDtypeStruct((NC, NPAD, D), jnp.float32),
    mesh=_mesh,
    scratch_types=[
        pltpu.VMEM((CH,), jnp.int32),
        pltpu.VMEM((CH,), jnp.int32),
        pltpu.VMEM((CH, D), jnp.float32),
        pltpu.VMEM((CH, D), jnp.float32),
        pltpu.VMEM_SHARED((NPAD, D), jnp.float32),
        pltpu.SemaphoreType.DMA,
        pltpu.SemaphoreType.DMA,
        pltpu.SemaphoreType.DMA,
        pltpu.SemaphoreType.DMA,
        pltpu.SemaphoreType.DMA,
        pltpu.SemaphoreType.DMA,
    ],
)
def _scatter(m_hbm, dst2d_hbm, out_hbm,
             idx0, idx1, rows0, rows1, accum,
             si0, si1, sr0, sr1, ss0, ss1):
    cid = lax.axis_index("c")
    sid = lax.axis_index("s")
    w = sid * NC + cid
    idxv = (idx0, idx1)
    rows = (rows0, rows1)
    si = (si0, si1)
    sr = (sr0, sr1)
    ss = (ss0, ss1)

    def zrow(i, carry):
        for l in range(D // L):
            rows0[i, pl.ds(l * L, L)] = jnp.zeros((L,), jnp.float32)
        return carry

    lax.fori_loop(0, ZR, zrow, 0)
    for k in range(NPT // ZR):
        pltpu.sync_copy(rows0, accum.at[pl.ds(sid * NPT + k * ZR, ZR)])
    plsc.subcore_barrier()

    def issue_loads(c, b):
        pltpu.async_copy(dst2d_hbm.at[c], idxv[b], si[b])
        pltpu.async_copy(m_hbm.at[pl.ds(c * CH, CH)], rows[b], sr[b])

    def wait_loads(b):
        pltpu.make_async_copy(dst2d_hbm.at[0], idxv[b], si[b]).wait()
        pltpu.make_async_copy(m_hbm.at[pl.ds(0, CH)], rows[b], sr[b]).wait()

    def wait_scatter(b):
        pltpu.make_async_copy(rows[b], accum.at[idxv[b]], ss[b]).wait()

    issue_loads(w, 0)

    def body(i, carry):
        for b in (0, 1):
            t = 2 * i + b
            nb = 1 - b
            c = w + t * NW

            @pl.when(t + 1 < FCHR)
            def _():
                @pl.when(t >= 1)
                def _():
                    wait_scatter(nb)
                issue_loads(c + NW, nb)

            wait_loads(b)
            pltpu.async_copy(rows[b], accum.at[idxv[b]], ss[b], add=True)
        return carry

    lax.fori_loop(0, FCHR // 2, body, 0)
    if FCHR % 2:
        b = (FCHR - 1) % 2
        wait_loads(b)
        pltpu.async_copy(rows[b], accum.at[idxv[b]], ss[b], add=True)
    wait_scatter(0)
    wait_scatter(1)

    @pl.when(w < TAILC)
    def _():
        c = w + FCHR * NW
        issue_loads(c, 0)
        wait_loads(0)
        pltpu.sync_copy(rows0, accum.at[idx0], add=True)

    plsc.subcore_barrier()

    for k in range(NPT // ZR):
        r0 = sid * NPT + k * ZR
        pltpu.sync_copy(accum.at[pl.ds(r0, ZR)], rows0)
        pltpu.sync_copy(rows0, out_hbm.at[cid, pl.ds(r0, ZR)])


def _upd_body(*refs):
    xu_ref = refs[0]
    p_refs = refs[1:1 + 2 * S]
    u1b_ref, u2_ref, ub2_ref, out_ref = refs[1 + 2 * S:]
    aggr = p_refs[0][...]
    for r in p_refs[1:]:
        aggr = aggr + r[...]
    h = jnp.maximum(
        xu_ref[...] + jnp.dot(aggr, u1b_ref[...],
                              preferred_element_type=jnp.float32), 0.0)
    out_ref[...] = jnp.dot(h, u2_ref[...],
                           preferred_element_type=jnp.float32) + ub2_ref[...]


def _update(xu, parts, u1b, u2, ub2):
    grid = (N // BN,)
    nd = pl.BlockSpec((BN, D), lambda i: (i, 0))
    wspec = pl.BlockSpec((D, D), lambda i: (0, 0))
    bspec = pl.BlockSpec((1, D), lambda i: (0, 0))
    return pl.pallas_call(
        _upd_body,
        grid=grid,
        in_specs=[nd] + [nd] * (2 * S) + [wspec, wspec, bspec],
        out_specs=nd,
        out_shape=jax.ShapeDtypeStruct((N, D), jnp.float32),
    )(xu, *parts, u1b, u2, ub2)


def kernel(x, edge_index, edge_attr, W1, b1, W2, b2, U1, ub1, U2, ub2):
    src = edge_index[0].astype(jnp.int32)
    dst = edge_index[1].astype(jnp.int32)
    wa, wb, w1c = W1[0:D], W1[D:2 * D], W1[2 * D:]
    ua, u1b = U1[0:D], U1[D:]
    b1r = b1.reshape(1, D)
    b2r = b2.reshape(1, D)
    ub1r = ub1.reshape(1, D)
    ub2r = ub2.reshape(1, D)

    xa, xb, xu = _precompute(x, wa, wb, ua, b1r, ub1r)

    parts = []
    for s in range(S):
        lo, hi = s * SE, (s + 1) * SE
        dst_s = dst[lo:hi]
        src_s = src[lo:hi]
        g = _gather(xa, xb, dst_s, src_s)
        m = _edge_mlp(g, edge_attr, w1c, W2, b2r, s)
        p = _scatter(m, dst_s.reshape(NCHS, CH))
        parts.append(p[0])
        parts.append(p[1])

    return _update(xu, parts, u1b, U2, ub2r)


# speedup vs baseline: 1.2898x; 1.0422x over previous
import functools

import jax
import jax.numpy as jnp
from jax import lax
from jax.experimental import pallas as pl
from jax.experimental.pallas import tpu as pltpu
from jax.experimental.pallas import tpu_sc as plsc

N = 10000
E = 320000
D = 128

NC, NS, L = 2, 16, 16
NW = NC * NS
CH = 128

S = 2
SE = E // S
EPW = SE // NW
FCH = EPW // CH
TAIL = EPW - FCH * CH
NCHS = SE // CH
FCHR = NCHS // NW
TAILC = NCHS - FCHR * NW

BN = 2000
BE = 2000

NPAD = 10240
NPT = NPAD // NS
ZR = 128

_mesh = plsc.VectorSubcoreMesh(core_axis_name="c", subcore_axis_name="s")


def _pre_body(x_ref, wa_ref, wb_ref, ua_ref, b1_ref, ub1_ref,
              xa_ref, xb_ref, xu_ref):
    x = x_ref[...]
    xa_ref[...] = jnp.dot(x, wa_ref[...], preferred_element_type=jnp.float32) + b1_ref[...]
    xb_ref[...] = jnp.dot(x, wb_ref[...], preferred_element_type=jnp.float32)
    xu_ref[...] = jnp.dot(x, ua_ref[...], preferred_element_type=jnp.float32) + ub1_ref[...]


def _precompute(x, wa, wb, ua, b1, ub1):
    grid = (N // BN,)
    return pl.pallas_call(
        _pre_body,
        grid=grid,
        in_specs=[
            pl.BlockSpec((BN, D), lambda i: (i, 0)),
            pl.BlockSpec((D, D), lambda i: (0, 0)),
            pl.BlockSpec((D, D), lambda i: (0, 0)),
            pl.BlockSpec((D, D), lambda i: (0, 0)),
            pl.BlockSpec((1, D), lambda i: (0, 0)),
            pl.BlockSpec((1, D), lambda i: (0, 0)),
        ],
        out_specs=[
            pl.BlockSpec((BN, D), lambda i: (i, 0)),
            pl.BlockSpec((BN, D), lambda i: (i, 0)),
            pl.BlockSpec((BN, D), lambda i: (i, 0)),
        ],
        out_shape=[jax.ShapeDtypeStruct((N, D), jnp.float32)] * 3,
    )(x, wa, wb, ua, b1, ub1)


def _vadd_rows(dst_buf, src_buf, nrows):
    def row(r, carry):
        for l in range(D // L):
            sl = pl.ds(l * L, L)
            dst_buf[r, sl] = dst_buf[r, sl] + src_buf[r, sl]
        return carry

    lax.fori_loop(0, nrows, row, 0)


def _make_gather(slab0):
    @functools.partial(
        pl.kernel,
        out_type=jax.ShapeDtypeStruct((SE, D), jnp.float32),
        mesh=_mesh,
        scratch_types=[
            pltpu.VMEM((EPW,), jnp.int32),
            pltpu.VMEM((EPW,), jnp.int32),
            pltpu.VMEM((CH, D), jnp.float32),
            pltpu.VMEM((CH, D), jnp.float32),
            pltpu.VMEM((CH, D), jnp.float32),
            pltpu.VMEM((CH, D), jnp.float32),
            pltpu.SemaphoreType.DMA,
            pltpu.SemaphoreType.DMA,
            pltpu.SemaphoreType.DMA,
            pltpu.SemaphoreType.DMA,
            pltpu.SemaphoreType.DMA,
            pltpu.SemaphoreType.DMA,
        ],
    )
    def _gather(xa_hbm, xb_hbm, ei_hbm, g_hbm,
                idxd, idxs, bufa0, bufa1, bufb0, bufb1,
                sa0, sa1, sb0, sb1, sw0, sw1):
        w = lax.axis_index("s") * NC + lax.axis_index("c")
        base = w * EPW
        bufa = (bufa0, bufa1)
        bufb = (bufb0, bufb1)
        sa = (sa0, sa1)
        sb = (sb0, sb1)
        sw = (sw0, sw1)

        pltpu.sync_copy(ei_hbm.at[pl.ds(E + slab0 + base, EPW)], idxd)
        pltpu.sync_copy(ei_hbm.at[pl.ds(slab0 + base, EPW)], idxs)

        def issue_gathers(t, b):
            off = t * CH
            pltpu.async_copy(xa_hbm.at[idxd.at[pl.ds(off, CH)]], bufa[b], sa[b])
            pltpu.async_copy(xb_hbm.at[idxs.at[pl.ds(off, CH)]], bufb[b], sb[b])

        def wait_gathers(b):
            pltpu.make_async_copy(
                xa_hbm.at[idxd.at[pl.ds(0, CH)]], bufa[b], sa[b]).wait()
            pltpu.make_async_copy(
                xb_hbm.at[idxs.at[pl.ds(0, CH)]], bufb[b], sb[b]).wait()

        def wait_write(b):
            pltpu.make_async_copy(bufa[b], g_hbm.at[pl.ds(base, CH)], sw[b]).wait()

        def process(t, b):
            wait_gathers(b)
            _vadd_rows(bufa[b], bufb[b], CH)
            pltpu.async_copy(bufa[b], g_hbm.at[pl.ds(base + t * CH, CH)], sw[b])

        issue_gathers(0, 0)

        def body(i, carry):
            for b in (0, 1):
                t = 2 * i + b
                nb = 1 - b

                @pl.when(t + 1 < FCH)
                def _():
                    @pl.when(t >= 1)
                    def _():
                        wait_write(nb)
                    issue_gathers(t + 1, nb)

                process(t, b)
            return carry

        lax.fori_loop(0, FCH // 2, body, 0)
        if FCH % 2:
            process(FCH - 1, (FCH - 1) % 2)

        wait_write(0)
        wait_write(1)

        toff = FCH * CH
        ta = pltpu.async_copy(
            xa_hbm.at[idxd.at[pl.ds(toff, TAIL)]], bufa0.at[pl.ds(0, TAIL)], sa0)
        tb = pltpu.async_copy(
            xb_hbm.at[idxs.at[pl.ds(toff, TAIL)]], bufb0.at[pl.ds(0, TAIL)], sb0)
        ta.wait()
        tb.wait()
        _vadd_rows(bufa0, bufb0, TAIL)
        pltpu.sync_copy(bufa0.at[pl.ds(0, TAIL)],
                        g_hbm.at[pl.ds(base + toff, TAIL)])

    return _gather


def _make_scatter(slab0):
    @functools.partial(
        pl.kernel,
        out_type=jax.ShapeDtypeStruct((NC, NPAD, D), jnp.float32),
        mesh=_mesh,
        scratch_types=[
            pltpu.VMEM((CH,), jnp.int32),
            pltpu.VMEM((CH,), jnp.int32),
            pltpu.VMEM((CH, D), jnp.float32),
            pltpu.VMEM((CH, D), jnp.float32),
            pltpu.VMEM_SHARED((NPAD, D), jnp.float32),
            pltpu.SemaphoreType.DMA,
            pltpu.SemaphoreType.DMA,
            pltpu.SemaphoreType.DMA,
            pltpu.SemaphoreType.DMA,
            pltpu.SemaphoreType.DMA,
            pltpu.SemaphoreType.DMA,
        ],
    )
    def _scatter(m_hbm, ei_hbm, out_hbm,
                 idx0, idx1, rows0, rows1, accum,
                 si0, si1, sr0, sr1, ss0, ss1):
        cid = lax.axis_index("c")
        sid = lax.axis_index("s")
        w = sid * NC + cid
        idxv = (idx0, idx1)
        rows = (rows0, rows1)
        si = (si0, si1)
        sr = (sr0, sr1)
        ss = (ss0, ss1)

        def zrow(i, carry):
            for l in range(D // L):
                rows0[i, pl.ds(l * L, L)] = jnp.zeros((L,), jnp.float32)
            return carry

        lax.fori_loop(0, ZR, zrow, 0)
        for k in range(NPT // ZR):
            pltpu.sync_copy(rows0, accum.at[pl.ds(sid * NPT + k * ZR, ZR)])
        plsc.subcore_barrier()

        def issue_loads(c, b):
            pltpu.async_copy(
                ei_hbm.at[pl.ds(E + slab0 + c * CH, CH)], idxv[b], si[b])
            pltpu.async_copy(m_hbm.at[pl.ds(c * CH, CH)], rows[b], sr[b])

        def wait_loads(b):
            pltpu.make_async_copy(
                ei_hbm.at[pl.ds(0, CH)], idxv[b], si[b]).wait()
            pltpu.make_async_copy(m_hbm.at[pl.ds(0, CH)], rows[b], sr[b]).wait()

        def wait_scatter(b):
            pltpu.make_async_copy(rows[b], accum.at[idxv[b]], ss[b]).wait()

        issue_loads(w, 0)

        def body(i, carry):
            for b in (0, 1):
                t = 2 * i + b
                nb = 1 - b
                c = w + t * NW

                @pl.when(t + 1 < FCHR)
                def _():
                    @pl.when(t >= 1)
                    def _():
                        wait_scatter(nb)
                    issue_loads(c + NW, nb)

                wait_loads(b)
                pltpu.async_copy(rows[b], accum.at[idxv[b]], ss[b], add=True)
            return carry

        lax.fori_loop(0, FCHR // 2, body, 0)
        if FCHR % 2:
            b = (FCHR - 1) % 2
            wait_loads(b)
            pltpu.async_copy(rows[b], accum.at[idxv[b]], ss[b], add=True)
        wait_scatter(0)
        wait_scatter(1)

        @pl.when(w < TAILC)
        def _():
            c = w + FCHR * NW
            issue_loads(c, 0)
            wait_loads(0)
            pltpu.sync_copy(rows0, accum.at[idx0], add=True)

        plsc.subcore_barrier()

        for k in range(NPT // ZR):
            r0 = sid * NPT + k * ZR
            pltpu.sync_copy(accum.at[pl.ds(r0, ZR)], rows0)
            pltpu.sync_copy(rows0, out_hbm.at[cid, pl.ds(r0, ZR)])

    return _scatter


_gathers = [_make_gather(s_ * SE) for s_ in range(S)]
_scatters = [_make_scatter(s_ * SE) for s_ in range(S)]


def _mlp_body(g_ref, ea_ref, w1c_ref, w2_ref, b2_ref, m_ref):
    m1 = g_ref[...] + jnp.dot(
        ea_ref[...], w1c_ref[...], preferred_element_type=jnp.float32)
    m_ref[...] = jnp.dot(jnp.maximum(m1, 0.0), w2_ref[...],
                         preferred_element_type=jnp.float32) + b2_ref[...]


def _edge_mlp(g, ea, w1c, w2, b2, slab):
    grid = (SE // BE,)
    off = slab * (SE // BE)
    return pl.pallas_call(
        _mlp_body,
        grid=grid,
        in_specs=[
            pl.BlockSpec((BE, D), lambda i: (i, 0)),
            pl.BlockSpec((BE, D), lambda i: (off + i, 0)),
            pl.BlockSpec((D, D), lambda i: (0, 0)),
            pl.BlockSpec((D, D), lambda i: (0, 0)),
            pl.BlockSpec((1, D), lambda i: (0, 0)),
        ],
        out_specs=pl.BlockSpec((BE, D), lambda i: (i, 0)),
        out_shape=jax.ShapeDtypeStruct((SE, D), jnp.float32),
    )(g, ea, w1c, w2, b2)


def _upd_body(*refs):
    xu_ref = refs[0]
    p_refs = refs[1:1 + 2 * S]
    u1b_ref, u2_ref, ub2_ref, out_ref = refs[1 + 2 * S:]
    aggr = p_refs[0][0]
    for r in p_refs[1:]:
        aggr = aggr + r[0]
    h = jnp.maximum(
        xu_ref[...] + jnp.dot(aggr, u1b_ref[...],
                              preferred_element_type=jnp.float32), 0.0)
    out_ref[...] = jnp.dot(h, u2_ref[...],
                           preferred_element_type=jnp.float32) + ub2_ref[...]


def _update(xu, parts, u1b, u2, ub2):
    grid = (N // BN,)
    nd = pl.BlockSpec((BN, D), lambda i: (i, 0))
    p0 = pl.BlockSpec((1, BN, D), lambda i: (0, i, 0))
    p1 = pl.BlockSpec((1, BN, D), lambda i: (1, i, 0))
    wspec = pl.BlockSpec((D, D), lambda i: (0, 0))
    bspec = pl.BlockSpec((1, D), lambda i: (0, 0))
    pin = []
    pargs = []
    for p in parts:
        pin += [p0, p1]
        pargs += [p, p]
    return pl.pallas_call(
        _upd_body,
        grid=grid,
        in_specs=[nd] + pin + [wspec, wspec, bspec],
        out_specs=nd,
        out_shape=jax.ShapeDtypeStruct((N, D), jnp.float32),
    )(xu, *pargs, u1b, u2, ub2)


def kernel(x, edge_index, edge_attr, W1, b1, W2, b2, U1, ub1, U2, ub2):
    ei = edge_index.astype(jnp.int32).reshape(2 * E)
    wa, wb, w1c = W1[0:D], W1[D:2 * D], W1[2 * D:]
    ua, u1b = U1[0:D], U1[D:]
    b1r = b1.reshape(1, D)
    b2r = b2.reshape(1, D)
    ub1r = ub1.reshape(1, D)
    ub2r = ub2.reshape(1, D)

    xa, xb, xu = _precompute(x, wa, wb, ua, b1r, ub1r)

    parts = []
    for s in range(S):
        g = _gathers[s](xa, xb, ei)
        m = _edge_mlp(g, edge_attr, w1c, W2, b2r, s)
        p = _scatters[s](m, ei)
        parts.append(p)

    return _update(xu, parts, u1b, U2, ub2r)


# speedup vs baseline: 1.2985x; 1.0067x over previous
import functools

import jax
import jax.numpy as jnp
from jax import lax
from jax.experimental import pallas as pl
from jax.experimental.pallas import tpu as pltpu
from jax.experimental.pallas import tpu_sc as plsc

N = 10000
E = 320000
D = 128

NC, NS, L = 2, 16, 16
NW = NC * NS
CH = 128

S = 2
SE = E // S
EPW = SE // NW
FCH = EPW // CH
TAIL = EPW - FCH * CH
NCHS = SE // CH
FCHR = NCHS // NW
TAILC = NCHS - FCHR * NW

BN = 2000
BE = 2000

NPAD = 10112
NPT = NPAD // NS
ZR = 128
_ZCHUNKS = [(0, 128), (128, 128), (256, 128), (384, 128), (512, 120)]

_mesh = plsc.VectorSubcoreMesh(core_axis_name="c", subcore_axis_name="s")


def _pre_body(x_ref, wa_ref, wb_ref, ua_ref, b1_ref, ub1_ref,
              xa_ref, xb_ref, xu_ref):
    x = x_ref[...]
    xa_ref[...] = jnp.dot(x, wa_ref[...], preferred_element_type=jnp.float32) + b1_ref[...]
    xb_ref[...] = jnp.dot(x, wb_ref[...], preferred_element_type=jnp.float32)
    xu_ref[...] = jnp.dot(x, ua_ref[...], preferred_element_type=jnp.float32) + ub1_ref[...]


def _precompute(x, wa, wb, ua, b1, ub1):
    grid = (N // BN,)
    return pl.pallas_call(
        _pre_body,
        grid=grid,
        in_specs=[
            pl.BlockSpec((BN, D), lambda i: (i, 0)),
            pl.BlockSpec((D, D), lambda i: (0, 0)),
            pl.BlockSpec((D, D), lambda i: (0, 0)),
            pl.BlockSpec((D, D), lambda i: (0, 0)),
            pl.BlockSpec((1, D), lambda i: (0, 0)),
            pl.BlockSpec((1, D), lambda i: (0, 0)),
        ],
        out_specs=[
            pl.BlockSpec((BN, D), lambda i: (i, 0)),
            pl.BlockSpec((BN, D), lambda i: (i, 0)),
            pl.BlockSpec((BN, D), lambda i: (i, 0)),
        ],
        out_shape=[jax.ShapeDtypeStruct((N, D), jnp.float32)] * 3,
    )(x, wa, wb, ua, b1, ub1)


def _vadd_rows(dst_buf, src_buf, nrows):
    def row(r, carry):
        for l in range(D // L):
            sl = pl.ds(l * L, L)
            dst_buf[r, sl] = dst_buf[r, sl] + src_buf[r, sl]
        return carry

    lax.fori_loop(0, nrows, row, 0)


def _make_gather(slab0):
    @functools.partial(
        pl.kernel,
        out_type=jax.ShapeDtypeStruct((SE, D), jnp.float32),
        mesh=_mesh,
        scratch_types=[
            pltpu.VMEM((EPW,), jnp.int32),
            pltpu.VMEM((EPW,), jnp.int32),
            pltpu.VMEM((CH, D), jnp.float32),
            pltpu.VMEM((CH, D), jnp.float32),
            pltpu.VMEM((CH, D), jnp.float32),
            pltpu.VMEM((CH, D), jnp.float32),
            pltpu.SemaphoreType.DMA,
            pltpu.SemaphoreType.DMA,
            pltpu.SemaphoreType.DMA,
            pltpu.SemaphoreType.DMA,
            pltpu.SemaphoreType.DMA,
            pltpu.SemaphoreType.DMA,
        ],
    )
    def _gather(xa_hbm, xb_hbm, ei_hbm, g_hbm,
                idxd, idxs, bufa0, bufa1, bufb0, bufb1,
                sa0, sa1, sb0, sb1, sw0, sw1):
        w = lax.axis_index("s") * NC + lax.axis_index("c")
        base = w * EPW
        bufa = (bufa0, bufa1)
        bufb = (bufb0, bufb1)
        sa = (sa0, sa1)
        sb = (sb0, sb1)
        sw = (sw0, sw1)

        pltpu.sync_copy(ei_hbm.at[pl.ds(E + slab0 + base, EPW)], idxd)
        pltpu.sync_copy(ei_hbm.at[pl.ds(slab0 + base, EPW)], idxs)

        def issue_gathers(t, b):
            off = t * CH
            pltpu.async_copy(xa_hbm.at[idxd.at[pl.ds(off, CH)]], bufa[b], sa[b])
            pltpu.async_copy(xb_hbm.at[idxs.at[pl.ds(off, CH)]], bufb[b], sb[b])

        def wait_gathers(b):
            pltpu.make_async_copy(
                xa_hbm.at[idxd.at[pl.ds(0, CH)]], bufa[b], sa[b]).wait()
            pltpu.make_async_copy(
                xb_hbm.at[idxs.at[pl.ds(0, CH)]], bufb[b], sb[b]).wait()

        def wait_write(b):
            pltpu.make_async_copy(bufa[b], g_hbm.at[pl.ds(base, CH)], sw[b]).wait()

        def process(t, b):
            wait_gathers(b)
            _vadd_rows(bufa[b], bufb[b], CH)
            pltpu.async_copy(bufa[b], g_hbm.at[pl.ds(base + t * CH, CH)], sw[b])

        issue_gathers(0, 0)

        def body(i, carry):
            for b in (0, 1):
                t = 2 * i + b
                nb = 1 - b

                @pl.when(t + 1 < FCH)
                def _():
                    @pl.when(t >= 1)
                    def _():
                        wait_write(nb)
                    issue_gathers(t + 1, nb)

                process(t, b)
            return carry

        lax.fori_loop(0, FCH // 2, body, 0)
        if FCH % 2:
            process(FCH - 1, (FCH - 1) % 2)

        wait_write(0)
        wait_write(1)

        toff = FCH * CH
        ta = pltpu.async_copy(
            xa_hbm.at[idxd.at[pl.ds(toff, TAIL)]], bufa0.at[pl.ds(0, TAIL)], sa0)
        tb = pltpu.async_copy(
            xb_hbm.at[idxs.at[pl.ds(toff, TAIL)]], bufb0.at[pl.ds(0, TAIL)], sb0)
        ta.wait()
        tb.wait()
        _vadd_rows(bufa0, bufb0, TAIL)
        pltpu.sync_copy(bufa0.at[pl.ds(0, TAIL)],
                        g_hbm.at[pl.ds(base + toff, TAIL)])

    return _gather


def _make_scatter(slab0):
    @functools.partial(
        pl.kernel,
        out_type=jax.ShapeDtypeStruct((NC, NPAD, D), jnp.float32),
        mesh=_mesh,
        scratch_types=[
            pltpu.VMEM((CH,), jnp.int32),
            pltpu.VMEM((CH,), jnp.int32),
            pltpu.VMEM((CH,), jnp.int32),
            pltpu.VMEM((CH, D), jnp.float32),
            pltpu.VMEM((CH, D), jnp.float32),
            pltpu.VMEM((CH, D), jnp.float32),
            pltpu.VMEM_SHARED((NPAD, D), jnp.float32),
            pltpu.SemaphoreType.DMA,
            pltpu.SemaphoreType.DMA,
            pltpu.SemaphoreType.DMA,
            pltpu.SemaphoreType.DMA,
            pltpu.SemaphoreType.DMA,
            pltpu.SemaphoreType.DMA,
            pltpu.SemaphoreType.DMA,
            pltpu.SemaphoreType.DMA,
            pltpu.SemaphoreType.DMA,
        ],
    )
    def _scatter(m_hbm, ei_hbm, out_hbm,
                 idx0, idx1, idx2, rows0, rows1, rows2, accum,
                 si0, si1, si2, sr0, sr1, sr2, ss0, ss1, ss2):
        cid = lax.axis_index("c")
        sid = lax.axis_index("s")
        w = sid * NC + cid
        idxv = (idx0, idx1, idx2)
        rows = (rows0, rows1, rows2)
        si = (si0, si1, si2)
        sr = (sr0, sr1, sr2)
        ss = (ss0, ss1, ss2)

        def zrow(i, carry):
            for l in range(D // L):
                rows0[i, pl.ds(l * L, L)] = jnp.zeros((L,), jnp.float32)
            return carry

        lax.fori_loop(0, ZR, zrow, 0)
        for off, ln in _ZCHUNKS:
            pltpu.sync_copy(rows0.at[pl.ds(0, ln)],
                            accum.at[pl.ds(sid * NPT + off, ln)])
        plsc.subcore_barrier()

        def issue_loads(c, b):
            pltpu.async_copy(
                ei_hbm.at[pl.ds(E + slab0 + c * CH, CH)], idxv[b], si[b])
            pltpu.async_copy(m_hbm.at[pl.ds(c * CH, CH)], rows[b], sr[b])

        def wait_loads(b):
            pltpu.make_async_copy(
                ei_hbm.at[pl.ds(0, CH)], idxv[b], si[b]).wait()
            pltpu.make_async_copy(m_hbm.at[pl.ds(0, CH)], rows[b], sr[b]).wait()

        def wait_scatter(b):
            pltpu.make_async_copy(rows[b], accum.at[idxv[b]], ss[b]).wait()

        issue_loads(w, 0)
        issue_loads(w + NW, 1)

        def body(i, carry):
            for b in (0, 1, 2):
                t = 3 * i + b
                nb = (b + 2) % 3
                c = w + t * NW

                @pl.when(t + 2 < FCHR)
                def _():
                    @pl.when(t >= 1)
                    def _():
                        wait_scatter(nb)
                    issue_loads(c + 2 * NW, nb)

                wait_loads(b)
                pltpu.async_copy(rows[b], accum.at[idxv[b]], ss[b], add=True)
            return carry

        lax.fori_loop(0, FCHR // 3, body, 0)
        wait_scatter(0)
        wait_scatter(1)
        wait_scatter(2)

        @pl.when(w < TAILC)
        def _():
            c = w + FCHR * NW
            issue_loads(c, 0)
            wait_loads(0)
            pltpu.sync_copy(rows0, accum.at[idx0], add=True)

        plsc.subcore_barrier()

        for off, ln in _ZCHUNKS:
            r0 = sid * NPT + off
            pltpu.sync_copy(accum.at[pl.ds(r0, ln)], rows0.at[pl.ds(0, ln)])
            pltpu.sync_copy(rows0.at[pl.ds(0, ln)], out_hbm.at[cid, pl.ds(r0, ln)])

    return _scatter


_gathers = [_make_gather(s_ * SE) for s_ in range(S)]
_scatters = [_make_scatter(s_ * SE) for s_ in range(S)]


def _mlp_body(g_ref, ea_ref, w1c_ref, w2_ref, b2_ref, m_ref):
    m1 = g_ref[...] + jnp.dot(
        ea_ref[...], w1c_ref[...], preferred_element_type=jnp.float32)
    m_ref[...] = jnp.dot(jnp.maximum(m1, 0.0), w2_ref[...],
                         preferred_element_type=jnp.float32) + b2_ref[...]


def _edge_mlp(g, ea, w1c, w2, b2, slab):
    grid = (SE // BE,)
    off = slab * (SE // BE)
    return pl.pallas_call(
        _mlp_body,
        grid=grid,
        in_specs=[
            pl.BlockSpec((BE, D), lambda i: (i, 0)),
            pl.BlockSpec((BE, D), lambda i: (off + i, 0)),
            pl.BlockSpec((D, D), lambda i: (0, 0)),
            pl.BlockSpec((D, D), lambda i: (0, 0)),
            pl.BlockSpec((1, D), lambda i: (0, 0)),
        ],
        out_specs=pl.BlockSpec((BE, D), lambda i: (i, 0)),
        out_shape=jax.ShapeDtypeStruct((SE, D), jnp.float32),
    )(g, ea, w1c, w2, b2)


def _upd_body(*refs):
    xu_ref = refs[0]
    p_refs = refs[1:1 + 2 * S]
    u1b_ref, u2_ref, ub2_ref, out_ref = refs[1 + 2 * S:]
    aggr = p_refs[0][0]
    for r in p_refs[1:]:
        aggr = aggr + r[0]
    h = jnp.maximum(
        xu_ref[...] + jnp.dot(aggr, u1b_ref[...],
                              preferred_element_type=jnp.float32), 0.0)
    out_ref[...] = jnp.dot(h, u2_ref[...],
                           preferred_element_type=jnp.float32) + ub2_ref[...]


def _update(xu, parts, u1b, u2, ub2):
    grid = (N // BN,)
    nd = pl.BlockSpec((BN, D), lambda i: (i, 0))
    p0 = pl.BlockSpec((1, BN, D), lambda i: (0, i, 0))
    p1 = pl.BlockSpec((1, BN, D), lambda i: (1, i, 0))
    wspec = pl.BlockSpec((D, D), lambda i: (0, 0))
    bspec = pl.BlockSpec((1, D), lambda i: (0, 0))
    pin = []
    pargs = []
    for p in parts:
        pin += [p0, p1]
        pargs += [p, p]
    return pl.pallas_call(
        _upd_body,
        grid=grid,
        in_specs=[nd] + pin + [wspec, wspec, bspec],
        out_specs=nd,
        out_shape=jax.ShapeDtypeStruct((N, D), jnp.float32),
    )(xu, *pargs, u1b, u2, ub2)


def kernel(x, edge_index, edge_attr, W1, b1, W2, b2, U1, ub1, U2, ub2):
    ei = edge_index.astype(jnp.int32).reshape(2 * E)
    wa, wb, w1c = W1[0:D], W1[D:2 * D], W1[2 * D:]
    ua, u1b = U1[0:D], U1[D:]
    b1r = b1.reshape(1, D)
    b2r = b2.reshape(1, D)
    ub1r = ub1.reshape(1, D)
    ub2r = ub2.reshape(1, D)

    xa, xb, xu = _precompute(x, wa, wb, ua, b1r, ub1r)

    parts = []
    for s in range(S):
        g = _gathers[s](xa, xb, ei)
        m = _edge_mlp(g, edge_attr, w1c, W2, b2r, s)
        p = _scatters[s](m, ei)
        parts.append(p)

    return _update(xu, parts, u1b, U2, ub2r)


# speedup vs baseline: 1.3456x; 1.0363x over previous
import functools

import jax
import jax.numpy as jnp
from jax import lax
from jax.experimental import pallas as pl
from jax.experimental.pallas import tpu as pltpu
from jax.experimental.pallas import tpu_sc as plsc

N = 10000
E = 320000
D = 128

NC, NS, L = 2, 16, 16
NW = NC * NS
CH = 128

S = 2
SE = E // S
EPW = SE // NW
FCH = EPW // CH
TAIL = EPW - FCH * CH
NCHS = SE // CH
FCHR = NCHS // NW
TAILC = NCHS - FCHR * NW

BN = 2000
BE = 4000

NPAD = 10112
NPT = NPAD // NS
ZR = 128
_ZCHUNKS = [(0, 128), (128, 128), (256, 128), (384, 128), (512, 120)]

_mesh = plsc.VectorSubcoreMesh(core_axis_name="c", subcore_axis_name="s")


def _pre_body(x_ref, wa_ref, wb_ref, ua_ref, b1_ref, ub1_ref,
              xa_ref, xb_ref, xu_ref):
    x = x_ref[...]
    xa_ref[...] = jnp.dot(x, wa_ref[...], preferred_element_type=jnp.float32) + b1_ref[...]
    xb_ref[...] = jnp.dot(x, wb_ref[...], preferred_element_type=jnp.float32)
    xu_ref[...] = jnp.dot(x, ua_ref[...], preferred_element_type=jnp.float32) + ub1_ref[...]


def _precompute(x, wa, wb, ua, b1, ub1):
    grid = (N // BN,)
    return pl.pallas_call(
        _pre_body,
        grid=grid,
        in_specs=[
            pl.BlockSpec((BN, D), lambda i: (i, 0)),
            pl.BlockSpec((D, D), lambda i: (0, 0)),
            pl.BlockSpec((D, D), lambda i: (0, 0)),
            pl.BlockSpec((D, D), lambda i: (0, 0)),
            pl.BlockSpec((1, D), lambda i: (0, 0)),
            pl.BlockSpec((1, D), lambda i: (0, 0)),
        ],
        out_specs=[
            pl.BlockSpec((BN, D), lambda i: (i, 0)),
            pl.BlockSpec((BN, D), lambda i: (i, 0)),
            pl.BlockSpec((BN, D), lambda i: (i, 0)),
        ],
        out_shape=[jax.ShapeDtypeStruct((N, D), jnp.float32)] * 3,
    )(x, wa, wb, ua, b1, ub1)


def _vadd_rows(dst_buf, src_buf, nrows):
    def row(r, carry):
        for l in range(D // L):
            sl = pl.ds(l * L, L)
            dst_buf[r, sl] = dst_buf[r, sl] + src_buf[r, sl]
        return carry

    lax.fori_loop(0, nrows, row, 0)


def _make_gather(slab0):
    @functools.partial(
        pl.kernel,
        out_type=jax.ShapeDtypeStruct((SE, D), jnp.float32),
        mesh=_mesh,
        scratch_types=[
            pltpu.VMEM((EPW,), jnp.int32),
            pltpu.VMEM((EPW,), jnp.int32),
            pltpu.VMEM((CH, D), jnp.float32),
            pltpu.VMEM((CH, D), jnp.float32),
            pltpu.VMEM((CH, D), jnp.float32),
            pltpu.VMEM((CH, D), jnp.float32),
            pltpu.SemaphoreType.DMA,
            pltpu.SemaphoreType.DMA,
            pltpu.SemaphoreType.DMA,
            pltpu.SemaphoreType.DMA,
            pltpu.SemaphoreType.DMA,
            pltpu.SemaphoreType.DMA,
        ],
    )
    def _gather(xa_hbm, xb_hbm, ei_hbm, g_hbm,
                idxd, idxs, bufa0, bufa1, bufb0, bufb1,
                sa0, sa1, sb0, sb1, sw0, sw1):
        w = lax.axis_index("s") * NC + lax.axis_index("c")
        base = w * EPW
        bufa = (bufa0, bufa1)
        bufb = (bufb0, bufb1)
        sa = (sa0, sa1)
        sb = (sb0, sb1)
        sw = (sw0, sw1)

        pltpu.sync_copy(ei_hbm.at[pl.ds(E + slab0 + base, EPW)], idxd)
        pltpu.sync_copy(ei_hbm.at[pl.ds(slab0 + base, EPW)], idxs)

        def issue_gathers(t, b):
            off = t * CH
            pltpu.async_copy(xa_hbm.at[idxd.at[pl.ds(off, CH)]], bufa[b], sa[b])
            pltpu.async_copy(xb_hbm.at[idxs.at[pl.ds(off, CH)]], bufb[b], sb[b])

        def wait_gathers(b):
            pltpu.make_async_copy(
                xa_hbm.at[idxd.at[pl.ds(0, CH)]], bufa[b], sa[b]).wait()
            pltpu.make_async_copy(
                xb_hbm.at[idxs.at[pl.ds(0, CH)]], bufb[b], sb[b]).wait()

        def wait_write(b):
            pltpu.make_async_copy(bufa[b], g_hbm.at[pl.ds(base, CH)], sw[b]).wait()

        def process(t, b):
            wait_gathers(b)
            _vadd_rows(bufa[b], bufb[b], CH)
            pltpu.async_copy(bufa[b], g_hbm.at[pl.ds(base + t * CH, CH)], sw[b])

        issue_gathers(0, 0)

        def body(i, carry):
            for b in (0, 1):
                t = 2 * i + b
                nb = 1 - b

                @pl.when(t + 1 < FCH)
                def _():
                    @pl.when(t >= 1)
                    def _():
                        wait_write(nb)
                    issue_gathers(t + 1, nb)

                process(t, b)
            return carry

        lax.fori_loop(0, FCH // 2, body, 0)
        if FCH % 2:
            process(FCH - 1, (FCH - 1) % 2)

        wait_write(0)
        wait_write(1)

        toff = FCH * CH
        ta = pltpu.async_copy(
            xa_hbm.at[idxd.at[pl.ds(toff, TAIL)]], bufa0.at[pl.ds(0, TAIL)], sa0)
        tb = pltpu.async_copy(
            xb_hbm.at[idxs.at[pl.ds(toff, TAIL)]], bufb0.at[pl.ds(0, TAIL)], sb0)
        ta.wait()
        tb.wait()
        _vadd_rows(bufa0, bufb0, TAIL)
        pltpu.sync_copy(bufa0.at[pl.ds(0, TAIL)],
                        g_hbm.at[pl.ds(base + toff, TAIL)])

    return _gather


def _make_scatter(slab0):
    @functools.partial(
        pl.kernel,
        out_type=jax.ShapeDtypeStruct((NC, NPAD, D), jnp.float32),
        mesh=_mesh,
        scratch_types=[
            pltpu.VMEM((CH,), jnp.int32),
            pltpu.VMEM((CH,), jnp.int32),
            pltpu.VMEM((CH,), jnp.int32),
            pltpu.VMEM((CH, D), jnp.float32),
            pltpu.VMEM((CH, D), jnp.float32),
            pltpu.VMEM((CH, D), jnp.float32),
            pltpu.VMEM_SHARED((NPAD, D), jnp.float32),
            pltpu.SemaphoreType.DMA,
            pltpu.SemaphoreType.DMA,
            pltpu.SemaphoreType.DMA,
            pltpu.SemaphoreType.DMA,
            pltpu.SemaphoreType.DMA,
            pltpu.SemaphoreType.DMA,
            pltpu.SemaphoreType.DMA,
            pltpu.SemaphoreType.DMA,
            pltpu.SemaphoreType.DMA,
        ],
    )
    def _scatter(m_hbm, ei_hbm, out_hbm,
                 idx0, idx1, idx2, rows0, rows1, rows2, accum,
                 si0, si1, si2, sr0, sr1, sr2, ss0, ss1, ss2):
        cid = lax.axis_index("c")
        sid = lax.axis_index("s")
        w = sid * NC + cid
        idxv = (idx0, idx1, idx2)
        rows = (rows0, rows1, rows2)
        si = (si0, si1, si2)
        sr = (sr0, sr1, sr2)
        ss = (ss0, ss1, ss2)

        def zrow(i, carry):
            for l in range(D // L):
                rows0[i, pl.ds(l * L, L)] = jnp.zeros((L,), jnp.float32)
            return carry

        lax.fori_loop(0, ZR, zrow, 0)
        for off, ln in _ZCHUNKS:
            pltpu.sync_copy(rows0.at[pl.ds(0, ln)],
                            accum.at[pl.ds(sid * NPT + off, ln)])
        plsc.subcore_barrier()

        def issue_loads(c, b):
            pltpu.async_copy(
                ei_hbm.at[pl.ds(E + slab0 + c * CH, CH)], idxv[b], si[b])
            pltpu.async_copy(m_hbm.at[pl.ds(c * CH, CH)], rows[b], sr[b])

        def wait_loads(b):
            pltpu.make_async_copy(
                ei_hbm.at[pl.ds(0, CH)], idxv[b], si[b]).wait()
            pltpu.make_async_copy(m_hbm.at[pl.ds(0, CH)], rows[b], sr[b]).wait()

        def wait_scatter(b):
            pltpu.make_async_copy(rows[b], accum.at[idxv[b]], ss[b]).wait()

        issue_loads(w, 0)
        issue_loads(w + NW, 1)

        def body(i, carry):
            for b in (0, 1, 2):
                t = 3 * i + b
                nb = (b + 2) % 3
                c = w + t * NW

                @pl.when(t + 2 < FCHR)
                def _():
                    @pl.when(t >= 1)
                    def _():
                        wait_scatter(nb)
                    issue_loads(c + 2 * NW, nb)

                wait_loads(b)
                pltpu.async_copy(rows[b], accum.at[idxv[b]], ss[b], add=True)
            return carry

        lax.fori_loop(0, FCHR // 3, body, 0)
        wait_scatter(0)
        wait_scatter(1)
        wait_scatter(2)

        @pl.when(w < TAILC)
        def _():
            c = w + FCHR * NW
            issue_loads(c, 0)
            wait_loads(0)
            pltpu.sync_copy(rows0, accum.at[idx0], add=True)

        plsc.subcore_barrier()

        for off, ln in _ZCHUNKS:
            r0 = sid * NPT + off
            pltpu.sync_copy(accum.at[pl.ds(r0, ln)], rows0.at[pl.ds(0, ln)])
            pltpu.sync_copy(rows0.at[pl.ds(0, ln)], out_hbm.at[cid, pl.ds(r0, ln)])

    return _scatter


_gathers = [_make_gather(s_ * SE) for s_ in range(S)]
_scatters = [_make_scatter(s_ * SE) for s_ in range(S)]


def _mlp_body(g_ref, ea_ref, w1c_ref, w2_ref, b2_ref, m_ref):
    m1 = g_ref[...] + jnp.dot(
        ea_ref[...], w1c_ref[...], preferred_element_type=jnp.float32)
    m_ref[...] = jnp.dot(jnp.maximum(m1, 0.0), w2_ref[...],
                         preferred_element_type=jnp.float32) + b2_ref[...]


def _edge_mlp(g, ea, w1c, w2, b2, slab):
    grid = (SE // BE,)
    off = slab * (SE // BE)
    return pl.pallas_call(
        _mlp_body,
        grid=grid,
        in_specs=[
            pl.BlockSpec((BE, D), lambda i: (i, 0)),
            pl.BlockSpec((BE, D), lambda i: (off + i, 0)),
            pl.BlockSpec((D, D), lambda i: (0, 0)),
            pl.BlockSpec((D, D), lambda i: (0, 0)),
            pl.BlockSpec((1, D), lambda i: (0, 0)),
        ],
        out_specs=pl.BlockSpec((BE, D), lambda i: (i, 0)),
        out_shape=jax.ShapeDtypeStruct((SE, D), jnp.float32),
    )(g, ea, w1c, w2, b2)


def _upd_body(*refs):
    xu_ref = refs[0]
    p_refs = refs[1:1 + 2 * S]
    u1b_ref, u2_ref, ub2_ref, out_ref = refs[1 + 2 * S:]
    aggr = p_refs[0][0]
    for r in p_refs[1:]:
        aggr = aggr + r[0]
    h = jnp.maximum(
        xu_ref[...] + jnp.dot(aggr, u1b_ref[...],
                              preferred_element_type=jnp.float32), 0.0)
    out_ref[...] = jnp.dot(h, u2_ref[...],
                           preferred_element_type=jnp.float32) + ub2_ref[...]


def _update(xu, parts, u1b, u2, ub2):
    grid = (N // BN,)
    nd = pl.BlockSpec((BN, D), lambda i: (i, 0))
    p0 = pl.BlockSpec((1, BN, D), lambda i: (0, i, 0))
    p1 = pl.BlockSpec((1, BN, D), lambda i: (1, i, 0))
    wspec = pl.BlockSpec((D, D), lambda i: (0, 0))
    bspec = pl.BlockSpec((1, D), lambda i: (0, 0))
    pin = []
    pargs = []
    for p in parts:
        pin += [p0, p1]
        pargs += [p, p]
    return pl.pallas_call(
        _upd_body,
        grid=grid,
        in_specs=[nd] + pin + [wspec, wspec, bspec],
        out_specs=nd,
        out_shape=jax.ShapeDtypeStruct((N, D), jnp.float32),
    )(xu, *pargs, u1b, u2, ub2)


def kernel(x, edge_index, edge_attr, W1, b1, W2, b2, U1, ub1, U2, ub2):
    ei = edge_index.astype(jnp.int32).reshape(2 * E)
    wa, wb, w1c = W1[0:D], W1[D:2 * D], W1[2 * D:]
    ua, u1b = U1[0:D], U1[D:]
    b1r = b1.reshape(1, D)
    b2r = b2.reshape(1, D)
    ub1r = ub1.reshape(1, D)
    ub2r = ub2.reshape(1, D)

    xa, xb, xu = _precompute(x, wa, wb, ua, b1r, ub1r)

    parts = []
    for s in range(S):
        g = _gathers[s](xa, xb, ei)
        m = _edge_mlp(g, edge_attr, w1c, W2, b2r, s)
        p = _scatters[s](m, ei)
        parts.append(p)

    return _update(xu, parts, u1b, U2, ub2r)
